# baseline (speedup 1.0000x reference)
import numpy as np

import jax
import jax.numpy as jnp
from jax import lax
from jax.experimental import pallas as pl
from jax.experimental.pallas import tpu as pltpu
from jax.experimental.pallas import tpu_sc as plsc

N_NODES = 10000
N_EDGES = 160000
NCH = 128

_z = np.linspace(-12.0, 12.0, 240001)
_pdf = np.exp(-0.5 * _z ** 2) / np.sqrt(2.0 * np.pi)
_sl = _z / (1.0 + np.exp(-_z))
_SILU_C = float(1.0 / np.sqrt(np.sum(_sl ** 2 * _pdf) * (_z[1] - _z[0])))


_EB = 2000


def _wt_body(len_ref, ea_ref, w1_ref, w2_ref, w3_ref, w4_ref, out_ref):
    hi = jax.lax.Precision.HIGHEST
    x = len_ref[...]
    x = _SILU_C * jax.nn.silu(
        jnp.dot(x, w1_ref[...] * (1.0 / np.sqrt(8.0)), precision=hi))
    x = _SILU_C * jax.nn.silu(
        jnp.dot(x, w2_ref[...] * (1.0 / np.sqrt(6.0)), precision=hi))
    x = _SILU_C * jax.nn.silu(
        jnp.dot(x, w3_ref[...] * (1.0 / np.sqrt(6.0)), precision=hi))
    tp = jnp.dot(x, w4_ref[...] * (1.0 / np.sqrt(6.0)), precision=hi)
    y = ea_ref[...]
    w0 = tp[:, :NCH]
    w1 = tp[:, NCH:]
    out_ref[0] = w0 * y[:, 0:1]
    out_ref[1] = w1 * y[:, 1:2]
    out_ref[2] = w1 * y[:, 2:3]
    out_ref[3] = w1 * y[:, 3:4]


def _compute_wt(length, edge_attributes, W1, W2, W3, W4):
    grid = N_EDGES // _EB
    return pl.pallas_call(
        _wt_body,
        out_shape=jax.ShapeDtypeStruct((4, N_EDGES, NCH), jnp.float32),
        grid=(grid,),
        in_specs=[
            pl.BlockSpec((_EB, 8), lambda i: (i, 0)),
            pl.BlockSpec((_EB, 4), lambda i: (i, 0)),
            pl.BlockSpec((8, 6), lambda i: (0, 0)),
            pl.BlockSpec((6, 6), lambda i: (0, 0)),
            pl.BlockSpec((6, 6), lambda i: (0, 0)),
            pl.BlockSpec((6, 256), lambda i: (0, 0)),
        ],
        out_specs=pl.BlockSpec((4, _EB, NCH), lambda i: (0, i, 0)),
    )(length, edge_attributes, W1, W2, W3, W4)


_NS = 16
_K = 16
_EPT = N_EDGES // _NS
_NB = _EPT // _K
_ZCH = 16
_NZCH = N_NODES // _ZCH
_FCH = 80
_NFCH = N_NODES // _FCH
_PIPELINED = True


def _sc_body(nf_hbm, wt_hbm, snd_hbm, rcv_hbm, out_hbm,
             idx_s, idx_r, xs, wt, mij0, mij1, acc,
             gsem0, gsem1, wsem0, wsem1, ssem0, ssem1):
    c = lax.axis_index("c")
    t = lax.axis_index("s")
    gsems = (gsem0, gsem1)
    wsems = (wsem0, wsem1)
    ssems = (ssem0, ssem1)
    mijs = (mij0, mij1)

    ebase = pl.multiple_of(t * _EPT, 8)
    pltpu.sync_copy(snd_hbm.at[pl.ds(ebase, _EPT)], idx_s)
    pltpu.sync_copy(rcv_hbm.at[pl.ds(ebase, _EPT)], idx_r)

    def _wait_into(dst_ref, sem):
        pltpu.make_async_copy(nf_hbm.at[pl.ds(0, _K)], dst_ref, sem).wait()

    def _gwait(s):
        iv = idx_s[pl.ds(0, _K)]
        pltpu.make_async_copy(nf_hbm.at[iv], xs.at[s], gsems[s]).wait()

    def _swait(s):
        iv = idx_r[pl.ds(0, _K)]
        pltpu.make_async_copy(mijs[s], acc.at[iv], ssems[s]).wait()

    def run_pass(p):
        plane = 2 * c + p
        wbase = plane * N_EDGES + t * _EPT

        @pl.loop(0, _K)
        def _zm(e):
            for s in range(2):
                for v in range(8):
                    mijs[s][e, pl.ds(v * 16, 16)] = jnp.zeros((16,), jnp.float32)

        @pl.loop(0, (_NZCH + _NS - 1) // _NS)
        def _za(j):
            cid = t + _NS * j

            @pl.when(cid < _NZCH)
            def _():
                pltpu.sync_copy(mij0, acc.at[pl.ds(cid * _ZCH, _ZCH)])

        plsc.subcore_barrier()

        idx0 = idx_r[pl.ds(0, _K)]
        pltpu.async_copy(mij0, acc.at[idx0], ssems[0], add=True)
        pltpu.async_copy(mij1, acc.at[idx0], ssems[1], add=True)

        def prime(b, s):
            sv = idx_s[pl.ds(pl.multiple_of(b * _K, 8), _K)]
            pltpu.async_copy(nf_hbm.at[sv], xs.at[s], gsems[s])
            woff = pl.multiple_of(wbase + b * _K, 8)
            pltpu.async_copy(wt_hbm.at[pl.ds(woff, _K)], wt.at[s], wsems[s])

        def compute(s):
            m = mijs[s]

            @pl.loop(0, _K, unroll=2)
            def _(e):
                for v in range(8):
                    sl = pl.ds(v * 16, 16)
                    m[e, sl] = xs[s, e, sl] * wt[s, e, sl]

        def scatter(b, s):
            rv = idx_r[pl.ds(pl.multiple_of(b * _K, 8), _K)]
            pltpu.async_copy(mijs[s], acc.at[rv], ssems[s], add=True)

        def step(b, s):
            _gwait(s)
            _wait_into(wt.at[s], wsems[s])
            _swait(s)
            compute(s)
            scatter(b, s)

        if _PIPELINED:
            prime(0, 0)

            @pl.loop(0, (_NB - 1) // 2)
            def _main(i):
                b0 = 2 * i
                prime(b0 + 1, 1)
                step(b0, 0)
                prime(b0 + 2, 0)
                step(b0 + 1, 1)

            step(_NB - 1, 0)
            _swait(0)
            _swait(1)
        else:
            _swait(0)
            _swait(1)

            @pl.loop(0, _NB)
            def _main(b):
                off = pl.multiple_of(b * _K, 8)
                sv = idx_s[pl.ds(off, _K)]
                pltpu.sync_copy(nf_hbm.at[sv], xs.at[0])
                woff = pl.multiple_of(wbase + b * _K, 8)
                pltpu.sync_copy(wt_hbm.at[pl.ds(woff, _K)], wt.at[0])
                compute(0)
                rv = idx_r[pl.ds(off, _K)]
                pltpu.sync_copy(mij0, acc.at[rv], add=True)

        plsc.subcore_barrier()

        @pl.loop(0, (_NFCH + _NS - 1) // _NS)
        def _fl(j):
            cid = t + _NS * j

            @pl.when(cid < _NFCH)
            def _():
                row = cid * _FCH
                pltpu.sync_copy(acc.at[pl.ds(row, _FCH)],
                                out_hbm.at[plane, pl.ds(row, _FCH)])

        plsc.subcore_barrier()

    run_pass(0)
    run_pass(1)


def _scatter_messages(node_features, wt, snd, rcv):
    mesh = plsc.VectorSubcoreMesh(core_axis_name="c", subcore_axis_name="s")
    return pl.kernel(
        _sc_body,
        out_type=jax.ShapeDtypeStruct((4, N_NODES, NCH), jnp.float32),
        mesh=mesh,
        scratch_types=[
            pltpu.VMEM((_EPT,), jnp.int32),
            pltpu.VMEM((_EPT,), jnp.int32),
            pltpu.VMEM((2, _K, NCH), jnp.float32),
            pltpu.VMEM((2, _K, NCH), jnp.float32),
            pltpu.VMEM((_K, NCH), jnp.float32),
            pltpu.VMEM((_K, NCH), jnp.float32),
            pltpu.VMEM_SHARED((N_NODES, NCH), jnp.float32),
            pltpu.SemaphoreType.DMA,
            pltpu.SemaphoreType.DMA,
            pltpu.SemaphoreType.DMA,
            pltpu.SemaphoreType.DMA,
            pltpu.SemaphoreType.DMA,
            pltpu.SemaphoreType.DMA,
        ],
    )(node_features, wt, snd, rcv)


_NBLK = 2000


def _mix_body(msg_ref, wl0_ref, wl1_ref, out_ref):
    hi = jax.lax.Precision.HIGHEST
    s = 1.0 / np.sqrt(float(NCH))
    out_ref[0] = jnp.dot(msg_ref[0], wl0_ref[...], precision=hi) * s
    out_ref[1] = jnp.dot(msg_ref[1], wl1_ref[...], precision=hi) * s
    out_ref[2] = jnp.dot(msg_ref[2], wl1_ref[...], precision=hi) * s
    out_ref[3] = jnp.dot(msg_ref[3], wl1_ref[...], precision=hi) * s


def _mix(msg, Wl0, Wl1):
    grid = N_NODES // _NBLK
    return pl.pallas_call(
        _mix_body,
        out_shape=jax.ShapeDtypeStruct((4, N_NODES, NCH), jnp.float32),
        grid=(grid,),
        in_specs=[
            pl.BlockSpec((4, _NBLK, NCH), lambda i: (0, i, 0)),
            pl.BlockSpec((NCH, NCH), lambda i: (0, 0)),
            pl.BlockSpec((NCH, NCH), lambda i: (0, 0)),
        ],
        out_specs=pl.BlockSpec((4, _NBLK, NCH), lambda i: (0, i, 0)),
    )(msg, Wl0, Wl1)


def kernel(length, node_features, edge_attributes, edge_index, W1, W2, W3, W4,
           Wl0, Wl1):
    wt = _compute_wt(length, edge_attributes, W1, W2, W3, W4)
    msg = _scatter_messages(node_features, wt.reshape(4 * N_EDGES, NCH),
                            edge_index[0], edge_index[1])
    out4 = _mix(msg, Wl0, Wl1)
    return jnp.transpose(out4, (1, 2, 0))


# speedup vs baseline: 1.0219x; 1.0219x over previous
import numpy as np

import jax
import jax.numpy as jnp
from jax import lax
from jax.experimental import pallas as pl
from jax.experimental.pallas import tpu as pltpu
from jax.experimental.pallas import tpu_sc as plsc

N_NODES = 10000
N_EDGES = 160000
NCH = 128

_z = np.linspace(-12.0, 12.0, 240001)
_pdf = np.exp(-0.5 * _z ** 2) / np.sqrt(2.0 * np.pi)
_sl = _z / (1.0 + np.exp(-_z))
_SILU_C = float(1.0 / np.sqrt(np.sum(_sl ** 2 * _pdf) * (_z[1] - _z[0])))


_EB = 2000


def _wt_body(len_ref, ea_ref, w1_ref, w2_ref, w3_ref, w4_ref, out_ref):
    hi = jax.lax.Precision.HIGHEST
    x = len_ref[...]
    x = _SILU_C * jax.nn.silu(
        jnp.dot(x, w1_ref[...] * (1.0 / np.sqrt(8.0)), precision=hi))
    x = _SILU_C * jax.nn.silu(
        jnp.dot(x, w2_ref[...] * (1.0 / np.sqrt(6.0)), precision=hi))
    x = _SILU_C * jax.nn.silu(
        jnp.dot(x, w3_ref[...] * (1.0 / np.sqrt(6.0)), precision=hi))
    tp = jnp.dot(x, w4_ref[...] * (1.0 / np.sqrt(6.0)), precision=hi)
    y = ea_ref[...]
    w0 = tp[:, :NCH]
    w1 = tp[:, NCH:]
    out_ref[0] = w0 * y[:, 0:1]
    out_ref[1] = w1 * y[:, 1:2]
    out_ref[2] = w1 * y[:, 2:3]
    out_ref[3] = w1 * y[:, 3:4]


def _compute_wt(length, edge_attributes, W1, W2, W3, W4):
    grid = N_EDGES // _EB
    return pl.pallas_call(
        _wt_body,
        out_shape=jax.ShapeDtypeStruct((4, N_EDGES, NCH), jnp.float32),
        grid=(grid,),
        in_specs=[
            pl.BlockSpec((_EB, 8), lambda i: (i, 0)),
            pl.BlockSpec((_EB, 4), lambda i: (i, 0)),
            pl.BlockSpec((8, 6), lambda i: (0, 0)),
            pl.BlockSpec((6, 6), lambda i: (0, 0)),
            pl.BlockSpec((6, 6), lambda i: (0, 0)),
            pl.BlockSpec((6, 256), lambda i: (0, 0)),
        ],
        out_specs=pl.BlockSpec((4, _EB, NCH), lambda i: (0, i, 0)),
    )(length, edge_attributes, W1, W2, W3, W4)


_NS = 16
_K = 16
_EPT = N_EDGES // _NS
_NB = _EPT // _K
_D = 4
_ZCH = 16
_NZCH = N_NODES // _ZCH
_FCH = 80
_NFCH = N_NODES // _FCH
_MASK16 = np.int32(0xFFFF)


def _sc_body(nf_hbm, wt_hbm, pk_hbm, out_hbm,
             idx_pk, xs, wt, mij0, mij1, mij2, mij3, acc,
             g0, g1, g2, g3, w0, w1, w2, w3, s0, s1, s2, s3):
    c = lax.axis_index("c")
    t = lax.axis_index("s")
    gsems = (g0, g1, g2, g3)
    wsems = (w0, w1, w2, w3)
    ssems = (s0, s1, s2, s3)
    mijs = (mij0, mij1, mij2, mij3)

    ebase = pl.multiple_of(t * _EPT, 8)
    pltpu.sync_copy(pk_hbm.at[pl.ds(ebase, _EPT)], idx_pk)

    def _sv(b):
        pv = idx_pk[pl.ds(pl.multiple_of(b * _K, 8), _K)]
        return jnp.bitwise_and(pv, _MASK16)

    def _rv(b):
        pv = idx_pk[pl.ds(pl.multiple_of(b * _K, 8), _K)]
        return lax.shift_right_logical(pv, 16)

    def _wait_into(dst_ref, sem):
        pltpu.make_async_copy(nf_hbm.at[pl.ds(0, _K)], dst_ref, sem).wait()

    def _gwait(s):
        iv = lax.iota(jnp.int32, 16)
        pltpu.make_async_copy(nf_hbm.at[iv], xs.at[s], gsems[s]).wait()

    def _swait(s):
        iv = lax.iota(jnp.int32, 16)
        pltpu.make_async_copy(mijs[s], acc.at[iv], ssems[s]).wait()

    def run_pass(p):
        plane = 2 * c + p
        wbase = plane * N_EDGES + t * _EPT

        @pl.loop(0, _K)
        def _zm(e):
            for s in range(_D):
                for v in range(8):
                    mijs[s][e, pl.ds(v * 16, 16)] = jnp.zeros((16,), jnp.float32)

        @pl.loop(0, (_NZCH + _NS - 1) // _NS)
        def _za(j):
            cid = t + _NS * j

            @pl.when(cid < _NZCH)
            def _():
                pltpu.sync_copy(mij0, acc.at[pl.ds(cid * _ZCH, _ZCH)])

        plsc.subcore_barrier()

        idx0 = _rv(0)
        for s in range(_D):
            pltpu.async_copy(mijs[s], acc.at[idx0], ssems[s], add=True)

        def prime(b, s):
            pltpu.async_copy(nf_hbm.at[_sv(b)], xs.at[s], gsems[s])
            woff = pl.multiple_of(wbase + b * _K, 8)
            pltpu.async_copy(wt_hbm.at[pl.ds(woff, _K)], wt.at[s], wsems[s])

        def compute(s):
            m = mijs[s]

            @pl.loop(0, _K, unroll=4)
            def _(e):
                for v in range(8):
                    sl = pl.ds(v * 16, 16)
                    m[e, sl] = xs[s, e, sl] * wt[s, e, sl]

        def step(b, s):
            _gwait(s)
            _wait_into(wt.at[s], wsems[s])
            _swait(s)
            compute(s)
            pltpu.async_copy(mijs[s], acc.at[_rv(b)], ssems[s], add=True)

        for b in range(_D - 1):
            prime(b, b)

        @pl.loop(0, (_NB - 1) // _D)
        def _main(i):
            for s in range(_D):
                b = _D * i + s
                nxt = b + _D - 1

                @pl.when(nxt < _NB)
                def _():
                    prime(nxt, (s + _D - 1) % _D)

                step(b, s)

        step(_NB - 1, (_NB - 1) % _D)
        for s in range(_D):
            _swait(s)

        plsc.subcore_barrier()

        @pl.loop(0, (_NFCH + _NS - 1) // _NS)
        def _fl(j):
            cid = t + _NS * j

            @pl.when(cid < _NFCH)
            def _():
                row = cid * _FCH
                pltpu.sync_copy(acc.at[pl.ds(row, _FCH)],
                                out_hbm.at[plane, pl.ds(row, _FCH)])

        plsc.subcore_barrier()

    run_pass(0)
    run_pass(1)


def _scatter_messages(node_features, wt, packed_idx):
    mesh = plsc.VectorSubcoreMesh(core_axis_name="c", subcore_axis_name="s")
    return pl.kernel(
        _sc_body,
        out_type=jax.ShapeDtypeStruct((4, N_NODES, NCH), jnp.float32),
        mesh=mesh,
        scratch_types=[
            pltpu.VMEM((_EPT,), jnp.int32),
            pltpu.VMEM((_D, _K, NCH), jnp.float32),
            pltpu.VMEM((_D, _K, NCH), jnp.float32),
            pltpu.VMEM((_K, NCH), jnp.float32),
            pltpu.VMEM((_K, NCH), jnp.float32),
            pltpu.VMEM((_K, NCH), jnp.float32),
            pltpu.VMEM((_K, NCH), jnp.float32),
            pltpu.VMEM_SHARED((N_NODES, NCH), jnp.float32),
        ] + [pltpu.SemaphoreType.DMA] * 12,
    )(node_features, wt, packed_idx)


_NBLK = 2000


def _mix_body(msg_ref, wl0_ref, wl1_ref, out_ref):
    hi = jax.lax.Precision.HIGHEST
    s = 1.0 / np.sqrt(float(NCH))
    out_ref[0] = jnp.dot(msg_ref[0], wl0_ref[...], precision=hi) * s
    out_ref[1] = jnp.dot(msg_ref[1], wl1_ref[...], precision=hi) * s
    out_ref[2] = jnp.dot(msg_ref[2], wl1_ref[...], precision=hi) * s
    out_ref[3] = jnp.dot(msg_ref[3], wl1_ref[...], precision=hi) * s


def _mix(msg, Wl0, Wl1):
    grid = N_NODES // _NBLK
    return pl.pallas_call(
        _mix_body,
        out_shape=jax.ShapeDtypeStruct((4, N_NODES, NCH), jnp.float32),
        grid=(grid,),
        in_specs=[
            pl.BlockSpec((4, _NBLK, NCH), lambda i: (0, i, 0)),
            pl.BlockSpec((NCH, NCH), lambda i: (0, 0)),
            pl.BlockSpec((NCH, NCH), lambda i: (0, 0)),
        ],
        out_specs=pl.BlockSpec((4, _NBLK, NCH), lambda i: (0, i, 0)),
    )(msg, Wl0, Wl1)


def kernel(length, node_features, edge_attributes, edge_index, W1, W2, W3, W4,
           Wl0, Wl1):
    wt = _compute_wt(length, edge_attributes, W1, W2, W3, W4)
    pk = edge_index[0] | (edge_index[1] << 16)
    msg = _scatter_messages(node_features, wt.reshape(4 * N_EDGES, NCH), pk)
    out4 = _mix(msg, Wl0, Wl1)
    return jnp.transpose(out4, (1, 2, 0))


# speedup vs baseline: 1.3688x; 1.3394x over previous
import numpy as np

import jax
import jax.numpy as jnp
from jax import lax
from jax.experimental import pallas as pl
from jax.experimental.pallas import tpu as pltpu
from jax.experimental.pallas import tpu_sc as plsc

N_NODES = 10000
N_EDGES = 160000
NCH = 128

_z = np.linspace(-12.0, 12.0, 240001)
_pdf = np.exp(-0.5 * _z ** 2) / np.sqrt(2.0 * np.pi)
_sl = _z / (1.0 + np.exp(-_z))
_SILU_C = float(1.0 / np.sqrt(np.sum(_sl ** 2 * _pdf) * (_z[1] - _z[0])))


_EB = 2000


def _wt_body(len_ref, ea_ref, w1_ref, w2_ref, w3_ref, w4_ref, out_ref):
    hi = None
    x = len_ref[...]
    x = _SILU_C * jax.nn.silu(
        jnp.dot(x, w1_ref[...] * (1.0 / np.sqrt(8.0)), precision=hi))
    x = _SILU_C * jax.nn.silu(
        jnp.dot(x, w2_ref[...] * (1.0 / np.sqrt(6.0)), precision=hi))
    x = _SILU_C * jax.nn.silu(
        jnp.dot(x, w3_ref[...] * (1.0 / np.sqrt(6.0)), precision=hi))
    tp = jnp.dot(x, w4_ref[...] * (1.0 / np.sqrt(6.0)), precision=hi)
    y = ea_ref[...]
    w0 = tp[:, :NCH]
    w1 = tp[:, NCH:]
    out_ref[0] = w0 * y[:, 0:1]
    out_ref[1] = w1 * y[:, 1:2]
    out_ref[2] = w1 * y[:, 2:3]
    out_ref[3] = w1 * y[:, 3:4]


def _compute_wt(length, edge_attributes, W1, W2, W3, W4):
    grid = N_EDGES // _EB
    return pl.pallas_call(
        _wt_body,
        out_shape=jax.ShapeDtypeStruct((4, N_EDGES, NCH), jnp.float32),
        grid=(grid,),
        in_specs=[
            pl.BlockSpec((_EB, 8), lambda i: (i, 0)),
            pl.BlockSpec((_EB, 4), lambda i: (i, 0)),
            pl.BlockSpec((8, 6), lambda i: (0, 0)),
            pl.BlockSpec((6, 6), lambda i: (0, 0)),
            pl.BlockSpec((6, 6), lambda i: (0, 0)),
            pl.BlockSpec((6, 256), lambda i: (0, 0)),
        ],
        out_specs=pl.BlockSpec((4, _EB, NCH), lambda i: (0, i, 0)),
    )(length, edge_attributes, W1, W2, W3, W4)


_NS = 16
_K = 16
_EPT = N_EDGES // _NS
_NB = _EPT // _K
_D = 4
_ZCH = 16
_NZCH = N_NODES // _ZCH
_FCH = 80
_NFCH = N_NODES // _FCH
_MASK16 = np.int32(0xFFFF)


def _sc_body(nf_hbm, wt_hbm, pk_hbm, out_hbm,
             idx_pk, xs, wt, mij0, mij1, mij2, mij3, acc,
             g0, g1, g2, g3, w0, w1, w2, w3, s0, s1, s2, s3):
    c = lax.axis_index("c")
    t = lax.axis_index("s")
    gsems = (g0, g1, g2, g3)
    wsems = (w0, w1, w2, w3)
    ssems = (s0, s1, s2, s3)
    mijs = (mij0, mij1, mij2, mij3)

    ebase = pl.multiple_of(t * _EPT, 8)
    pltpu.sync_copy(pk_hbm.at[pl.ds(ebase, _EPT)], idx_pk)

    def _sv(b):
        pv = idx_pk[pl.ds(pl.multiple_of(b * _K, 8), _K)]
        return jnp.bitwise_and(pv, _MASK16)

    def _rv(b):
        pv = idx_pk[pl.ds(pl.multiple_of(b * _K, 8), _K)]
        return lax.shift_right_logical(pv, 16)

    def _wait_into(dst_ref, sem):
        pltpu.make_async_copy(nf_hbm.at[pl.ds(0, _K)], dst_ref, sem).wait()

    def _gwait(s):
        iv = lax.iota(jnp.int32, 16)
        pltpu.make_async_copy(nf_hbm.at[iv], xs.at[s], gsems[s]).wait()

    def _swait(s):
        iv = lax.iota(jnp.int32, 16)
        pltpu.make_async_copy(mijs[s], acc.at[iv], ssems[s]).wait()

    def run_pass(p):
        plane = 2 * c + p
        wbase = plane * N_EDGES + t * _EPT

        @pl.loop(0, _K)
        def _zm(e):
            for s in range(_D):
                for v in range(8):
                    mijs[s][e, pl.ds(v * 16, 16)] = jnp.zeros((16,), jnp.float32)

        @pl.loop(0, (_NZCH + _NS - 1) // _NS)
        def _za(j):
            cid = t + _NS * j

            @pl.when(cid < _NZCH)
            def _():
                pltpu.sync_copy(mij0, acc.at[pl.ds(cid * _ZCH, _ZCH)])

        plsc.subcore_barrier()

        idx0 = _rv(0)
        for s in range(_D):
            pltpu.async_copy(mijs[s], acc.at[idx0], ssems[s], add=True)

        def prime(b, s):
            pltpu.async_copy(nf_hbm.at[_sv(b)], xs.at[s], gsems[s])
            woff = pl.multiple_of(wbase + b * _K, 8)
            pltpu.async_copy(wt_hbm.at[pl.ds(woff, _K)], wt.at[s], wsems[s])

        def compute(s):
            m = mijs[s]

            @pl.loop(0, _K, unroll=4)
            def _(e):
                for v in range(8):
                    sl = pl.ds(v * 16, 16)
                    m[e, sl] = xs[s, e, sl] * wt[s, e, sl]

        def step(b, s):
            _gwait(s)
            _wait_into(wt.at[s], wsems[s])
            _swait(s)
            compute(s)
            pltpu.async_copy(mijs[s], acc.at[_rv(b)], ssems[s], add=True)

        for b in range(_D - 1):
            prime(b, b)

        @pl.loop(0, (_NB - 1) // _D)
        def _main(i):
            for s in range(_D):
                b = _D * i + s
                nxt = b + _D - 1

                @pl.when(nxt < _NB)
                def _():
                    prime(nxt, (s + _D - 1) % _D)

                step(b, s)

        step(_NB - 1, (_NB - 1) % _D)
        for s in range(_D):
            _swait(s)

        plsc.subcore_barrier()

        @pl.loop(0, (_NFCH + _NS - 1) // _NS)
        def _fl(j):
            cid = t + _NS * j

            @pl.when(cid < _NFCH)
            def _():
                row = cid * _FCH
                pltpu.sync_copy(acc.at[pl.ds(row, _FCH)],
                                out_hbm.at[plane, pl.ds(row, _FCH)])

        plsc.subcore_barrier()

    run_pass(0)
    run_pass(1)


def _scatter_messages(node_features, wt, packed_idx):
    mesh = plsc.VectorSubcoreMesh(core_axis_name="c", subcore_axis_name="s")
    return pl.kernel(
        _sc_body,
        out_type=jax.ShapeDtypeStruct((4, N_NODES, NCH), jnp.float32),
        mesh=mesh,
        scratch_types=[
            pltpu.VMEM((_EPT,), jnp.int32),
            pltpu.VMEM((_D, _K, NCH), jnp.float32),
            pltpu.VMEM((_D, _K, NCH), jnp.float32),
            pltpu.VMEM((_K, NCH), jnp.float32),
            pltpu.VMEM((_K, NCH), jnp.float32),
            pltpu.VMEM((_K, NCH), jnp.float32),
            pltpu.VMEM((_K, NCH), jnp.float32),
            pltpu.VMEM_SHARED((N_NODES, NCH), jnp.float32),
        ] + [pltpu.SemaphoreType.DMA] * 12,
    )(node_features, wt, packed_idx)


_NBLK = 2000


def _mix_body(msg_ref, wl0_ref, wl1_ref, out_ref):
    hi = None
    s = 1.0 / np.sqrt(float(NCH))
    out_ref[0] = jnp.dot(msg_ref[0], wl0_ref[...], precision=hi) * s
    out_ref[1] = jnp.dot(msg_ref[1], wl1_ref[...], precision=hi) * s
    out_ref[2] = jnp.dot(msg_ref[2], wl1_ref[...], precision=hi) * s
    out_ref[3] = jnp.dot(msg_ref[3], wl1_ref[...], precision=hi) * s


def _mix(msg, Wl0, Wl1):
    grid = N_NODES // _NBLK
    return pl.pallas_call(
        _mix_body,
        out_shape=jax.ShapeDtypeStruct((4, N_NODES, NCH), jnp.float32),
        grid=(grid,),
        in_specs=[
            pl.BlockSpec((4, _NBLK, NCH), lambda i: (0, i, 0)),
            pl.BlockSpec((NCH, NCH), lambda i: (0, 0)),
            pl.BlockSpec((NCH, NCH), lambda i: (0, 0)),
        ],
        out_specs=pl.BlockSpec((4, _NBLK, NCH), lambda i: (0, i, 0)),
    )(msg, Wl0, Wl1)


def kernel(length, node_features, edge_attributes, edge_index, W1, W2, W3, W4,
           Wl0, Wl1):
    wt = _compute_wt(length, edge_attributes, W1, W2, W3, W4)
    pk = edge_index[0] | (edge_index[1] << 16)
    msg = _scatter_messages(node_features, wt.reshape(4 * N_EDGES, NCH), pk)
    out4 = _mix(msg, Wl0, Wl1)
    return jnp.transpose(out4, (1, 2, 0))
